# Initial kernel scaffold; baseline (speedup 1.0000x reference)
import jax
import jax.numpy as jnp
from jax.experimental import pallas as pl


def kernel(xyz, xyz_embed, new_xyz, new_xyz_embed, features):
    raise NotImplementedError("write your pallas kernel here")


# speedup vs baseline: 5.8320x; 5.8320x over previous
import jax
import jax.numpy as jnp
from jax import lax
from jax.experimental import pallas as pl
from jax.experimental.pallas import tpu as pltpu
from jax.experimental.pallas import tpu_sc as plsc

_RADIUS = 0.2
_NSAMPLE = 32


def _build_table(xyz_embed, features):
    B, N, EMB = xyz_embed.shape
    C = features.shape[1]
    D = EMB + C
    TN = 512
    n_blk = N // TN

    def body(emb_ref, feat_ref, out_ref):
        out_ref[...] = jnp.concatenate([emb_ref[0], feat_ref[0].T], axis=1)

    return pl.pallas_call(
        body,
        grid=(B, n_blk),
        in_specs=[
            pl.BlockSpec((1, TN, EMB), lambda b, i: (b, i, 0)),
            pl.BlockSpec((1, C, TN), lambda b, i: (b, 0, i)),
        ],
        out_specs=pl.BlockSpec((TN, D), lambda b, i: (b * n_blk + i, 0)),
        out_shape=jax.ShapeDtypeStruct((B * N, D), jnp.float32),
    )(xyz_embed, features)


def _ball_query_sc(xyz, new_xyz):
    B, N, _ = xyz.shape
    NP = new_xyz.shape[1]
    NS = _NSAMPLE
    r2 = _RADIUS * _RADIUS

    info = plsc.get_sparse_core_info()
    NC, NSUB, L = info.num_cores, info.num_subcores, info.num_lanes
    NW = NC * NSUB
    CPW = (B * NP) // NW
    TOT = B * NP * NS
    n_chunks = N // L

    mesh = plsc.VectorSubcoreMesh(core_axis_name="c", subcore_axis_name="s")

    def body(xyz_hbm, new_hbm, out_hbm, xyz_v, new_v, idxbuf, outbuf):
        cid = lax.axis_index("c")
        sid = lax.axis_index("s")
        wid = sid * NC + cid
        g0 = wid * CPW
        b = g0 // NP
        p0 = g0 % NP
        pltpu.sync_copy(xyz_hbm.at[b], xyz_v)
        pltpu.sync_copy(new_hbm.at[b, pl.ds(p0 * 3, CPW * 3)], new_v)
        bN = b * N
        lanes = lax.broadcasted_iota(jnp.int32, (L,), 0)
        zeros = jnp.zeros((L,), jnp.int32)

        def per_centroid(k, carry):
            qbase = zeros + k * 3
            qx = plsc.load_gather(new_v, [qbase])
            qy = plsc.load_gather(new_v, [qbase + 1])
            qz = plsc.load_gather(new_v, [qbase + 2])
            idxbuf[pl.ds(0, L)] = jnp.full((L,), bN, jnp.int32)

            def cond(jc):
                j, cnt = jc
                return jnp.logical_and(j < n_chunks, cnt < NS)

            def step(jc):
                j, cnt = jc
                nidx = lanes + j * L
                nidx3 = nidx * 3
                px = plsc.load_gather(xyz_v, [nidx3])
                py = plsc.load_gather(xyz_v, [nidx3 + 1])
                pz = plsc.load_gather(xyz_v, [nidx3 + 2])
                dx = px - qx
                dy = py - qy
                dz = pz - qz
                d2 = dx * dx + dy * dy + dz * dz
                m = d2 <= r2
                plsc.store_compressed(idxbuf.at[pl.ds(cnt, L)], nidx + bN, mask=m)
                return j + 1, cnt + jnp.sum(m.astype(jnp.int32))

            _, total = lax.while_loop(cond, step, (jnp.int32(0), jnp.int32(0)))
            v0 = idxbuf[pl.ds(0, L)]
            v1 = idxbuf[pl.ds(L, L)]
            fvec = jnp.full((L,), v0[0], jnp.int32)
            outbuf[pl.ds(k * NS, L)] = jnp.where(lanes < total, v0, fvec)
            outbuf[pl.ds(k * NS + L, L)] = jnp.where(lanes + L < total, v1, fvec)
            return carry

        lax.fori_loop(0, CPW, per_centroid, 0)
        pltpu.sync_copy(outbuf, out_hbm.at[pl.ds(g0 * NS, CPW * NS)])

    bq = pl.kernel(
        body,
        out_type=jax.ShapeDtypeStruct((TOT,), jnp.int32),
        mesh=mesh,
        compiler_params=pltpu.CompilerParams(needs_layout_passes=False),
        scratch_types=[
            pltpu.VMEM((N * 3,), jnp.float32),
            pltpu.VMEM((CPW * 3,), jnp.float32),
            pltpu.VMEM((4 * L,), jnp.int32),
            pltpu.VMEM((CPW * NS,), jnp.int32),
        ],
    )
    return bq(xyz.reshape(B, N * 3), new_xyz.reshape(B, NP * 3))


def _gather_sc(table, idx_flat):
    TOT = idx_flat.shape[0]
    D = table.shape[1]

    info = plsc.get_sparse_core_info()
    NC, NSUB = info.num_cores, info.num_subcores
    NW = NC * NSUB
    PW = TOT // NW
    CH = 128
    n_chunks = PW // CH

    mesh = plsc.VectorSubcoreMesh(core_axis_name="c", subcore_axis_name="s")

    def body(table_hbm, idx_hbm, out_hbm, idx_v, rows, sem):
        cid = lax.axis_index("c")
        sid = lax.axis_index("s")
        wid = sid * NC + cid
        base = wid * PW
        pltpu.sync_copy(idx_hbm.at[pl.ds(base, PW)], idx_v)

        def chunk(i, carry):
            off = i * CH
            pltpu.async_copy(table_hbm.at[idx_v.at[pl.ds(off, CH)]], rows, sem).wait()
            pltpu.sync_copy(rows, out_hbm.at[pl.ds(base + off, CH)])
            return carry

        lax.fori_loop(0, n_chunks, chunk, 0)

    g = pl.kernel(
        body,
        out_type=jax.ShapeDtypeStruct((TOT, D), jnp.float32),
        mesh=mesh,
        compiler_params=pltpu.CompilerParams(use_tc_tiling_on_sc=False),
        scratch_types=[
            pltpu.VMEM((PW,), jnp.int32),
            pltpu.VMEM((CH, D), jnp.float32),
            pltpu.SemaphoreType.DMA,
        ],
    )
    return g(table, idx_flat)


def _untranspose(gathered, new_xyz_embed, NP):
    TOT, D = gathered.shape
    B, _, EMB = new_xyz_embed.shape
    NS = _NSAMPLE
    NPS = NP * NS
    PS = 256
    PPER = PS // NS
    n_blk = NPS // PS

    def body(g_ref, emb_ref, out_ref):
        gt = g_ref[...].T
        e = emb_ref[0]
        sel = (lax.broadcasted_iota(jnp.int32, (PPER, PS), 1) // NS ==
               lax.broadcasted_iota(jnp.int32, (PPER, PS), 0)).astype(jnp.float32)
        et = jax.lax.dot(e.T, sel, preferred_element_type=jnp.float32)
        out_ref[0, :EMB, :] = gt[:EMB, :] - et
        out_ref[0, EMB:, :] = gt[EMB:, :]

    out2 = pl.pallas_call(
        body,
        grid=(B, n_blk),
        in_specs=[
            pl.BlockSpec((PS, D), lambda b, i: (b * n_blk + i, 0)),
            pl.BlockSpec((1, PPER, EMB), lambda b, i: (b, i, 0)),
        ],
        out_specs=pl.BlockSpec((1, D, PS), lambda b, i: (b, 0, i)),
        out_shape=jax.ShapeDtypeStruct((B, D, NPS), jnp.float32),
    )(gathered, new_xyz_embed)
    return out2.reshape(B, D, NP, NS)


def kernel(xyz, xyz_embed, new_xyz, new_xyz_embed, features):
    NP = new_xyz.shape[1]
    table = _build_table(xyz_embed, features)
    idx_flat = _ball_query_sc(xyz, new_xyz)
    gathered = _gather_sc(table, idx_flat)
    return _untranspose(gathered, new_xyz_embed, NP)


# speedup vs baseline: 6.6314x; 1.1371x over previous
import jax
import jax.numpy as jnp
from jax import lax
from jax.experimental import pallas as pl
from jax.experimental.pallas import tpu as pltpu
from jax.experimental.pallas import tpu_sc as plsc

_RADIUS = 0.2
_NSAMPLE = 32


def _build_table(xyz_embed, features):
    B, N, EMB = xyz_embed.shape
    C = features.shape[1]
    D = EMB + C
    DP = 384
    TN = 512
    n_blk = N // TN

    def body(emb_ref, feat_ref, out_ref):
        pad = jnp.zeros((TN, DP - D), jnp.float32)
        out_ref[...] = jnp.concatenate([emb_ref[0], feat_ref[0].T, pad], axis=1)

    return pl.pallas_call(
        body,
        grid=(B, n_blk),
        in_specs=[
            pl.BlockSpec((1, TN, EMB), lambda b, i: (b, i, 0)),
            pl.BlockSpec((1, C, TN), lambda b, i: (b, 0, i)),
        ],
        out_specs=pl.BlockSpec((TN, DP), lambda b, i: (b * n_blk + i, 0)),
        out_shape=jax.ShapeDtypeStruct((B * N, DP), jnp.float32),
    )(xyz_embed, features)


def _ball_query_sc(xyz, new_xyz):
    B, N, _ = xyz.shape
    NP = new_xyz.shape[1]
    NS = _NSAMPLE
    r2 = _RADIUS * _RADIUS

    info = plsc.get_sparse_core_info()
    NC, NSUB, L = info.num_cores, info.num_subcores, info.num_lanes
    NW = NC * NSUB
    CPW = (B * NP) // NW
    TOT = B * NP * NS
    n_chunks = N // L

    mesh = plsc.VectorSubcoreMesh(core_axis_name="c", subcore_axis_name="s")

    def body(xyz_hbm, new_hbm, out_hbm, xyz_v, new_v, idxbuf, outbuf):
        cid = lax.axis_index("c")
        sid = lax.axis_index("s")
        wid = sid * NC + cid
        g0 = wid * CPW
        b = g0 // NP
        p0 = g0 % NP
        pltpu.sync_copy(xyz_hbm.at[b], xyz_v)
        pltpu.sync_copy(new_hbm.at[b, pl.ds(p0 * 3, CPW * 3)], new_v)
        bN = b * N
        lanes = lax.broadcasted_iota(jnp.int32, (L,), 0)
        zeros = jnp.zeros((L,), jnp.int32)

        def per_centroid(k, carry):
            qbase = zeros + k * 3
            qx = plsc.load_gather(new_v, [qbase])
            qy = plsc.load_gather(new_v, [qbase + 1])
            qz = plsc.load_gather(new_v, [qbase + 2])
            idxbuf[pl.ds(0, L)] = jnp.full((L,), bN, jnp.int32)

            def cond(jc):
                j, cnt = jc
                return jnp.logical_and(j < n_chunks, cnt < NS)

            def step(jc):
                j, cnt = jc
                nidx = lanes + j * L
                nidx3 = nidx * 3
                px = plsc.load_gather(xyz_v, [nidx3])
                py = plsc.load_gather(xyz_v, [nidx3 + 1])
                pz = plsc.load_gather(xyz_v, [nidx3 + 2])
                dx = px - qx
                dy = py - qy
                dz = pz - qz
                d2 = dx * dx + dy * dy + dz * dz
                m = d2 <= r2
                plsc.store_compressed(idxbuf.at[pl.ds(cnt, L)], nidx + bN, mask=m)
                return j + 1, cnt + jnp.sum(m.astype(jnp.int32))

            _, total = lax.while_loop(cond, step, (jnp.int32(0), jnp.int32(0)))
            v0 = idxbuf[pl.ds(0, L)]
            v1 = idxbuf[pl.ds(L, L)]
            fvec = jnp.full((L,), v0[0], jnp.int32)
            outbuf[pl.ds(k * NS, L)] = jnp.where(lanes < total, v0, fvec)
            outbuf[pl.ds(k * NS + L, L)] = jnp.where(lanes + L < total, v1, fvec)
            return carry

        lax.fori_loop(0, CPW, per_centroid, 0)
        pltpu.sync_copy(outbuf, out_hbm.at[pl.ds(g0 * NS, CPW * NS)])

    bq = pl.kernel(
        body,
        out_type=jax.ShapeDtypeStruct((TOT,), jnp.int32),
        mesh=mesh,
        compiler_params=pltpu.CompilerParams(needs_layout_passes=False),
        scratch_types=[
            pltpu.VMEM((N * 3,), jnp.float32),
            pltpu.VMEM((CPW * 3,), jnp.float32),
            pltpu.VMEM((4 * L,), jnp.int32),
            pltpu.VMEM((CPW * NS,), jnp.int32),
        ],
    )
    return bq(xyz.reshape(B, N * 3), new_xyz.reshape(B, NP * 3))


def _gather_sc(table, idx_flat):
    TOT = idx_flat.shape[0]
    D = table.shape[1]

    info = plsc.get_sparse_core_info()
    NC, NSUB = info.num_cores, info.num_subcores
    NW = NC * NSUB
    PW = TOT // NW
    CH = 128
    n_chunks = PW // CH

    mesh = plsc.VectorSubcoreMesh(core_axis_name="c", subcore_axis_name="s")

    def body(table_hbm, idx_hbm, out_hbm, idx_v, rows, sem):
        cid = lax.axis_index("c")
        sid = lax.axis_index("s")
        wid = sid * NC + cid
        base = wid * PW
        pltpu.sync_copy(idx_hbm.at[pl.ds(base, PW)], idx_v)

        def chunk(i, carry):
            off = i * CH
            pltpu.async_copy(table_hbm.at[idx_v.at[pl.ds(off, CH)]], rows, sem).wait()
            pltpu.sync_copy(rows, out_hbm.at[pl.ds(base + off, CH)])
            return carry

        lax.fori_loop(0, n_chunks, chunk, 0)

    g = pl.kernel(
        body,
        out_type=jax.ShapeDtypeStruct((TOT, D), jnp.float32),
        mesh=mesh,
        scratch_types=[
            pltpu.VMEM((PW,), jnp.int32),
            pltpu.VMEM((CH, D), jnp.float32),
            pltpu.SemaphoreType.DMA,
        ],
    )
    return g(table, idx_flat)


def _untranspose(gathered, new_xyz_embed, NP, D):
    TOT, DP = gathered.shape
    B, _, EMB = new_xyz_embed.shape
    NS = _NSAMPLE
    NPS = NP * NS
    PS = 256
    PPER = PS // NS
    n_blk = NPS // PS

    def body(g_ref, emb_ref, out_ref):
        gt = g_ref[:, :D].T
        e = emb_ref[0]
        sel = (lax.broadcasted_iota(jnp.int32, (PPER, PS), 1) // NS ==
               lax.broadcasted_iota(jnp.int32, (PPER, PS), 0)).astype(jnp.float32)
        et = jax.lax.dot(e.T, sel, preferred_element_type=jnp.float32)
        out_ref[0, :EMB, :] = gt[:EMB, :] - et
        out_ref[0, EMB:, :] = gt[EMB:, :]

    out2 = pl.pallas_call(
        body,
        grid=(B, n_blk),
        in_specs=[
            pl.BlockSpec((PS, DP), lambda b, i: (b * n_blk + i, 0)),
            pl.BlockSpec((1, PPER, EMB), lambda b, i: (b, i, 0)),
        ],
        out_specs=pl.BlockSpec((1, D, PS), lambda b, i: (b, 0, i)),
        out_shape=jax.ShapeDtypeStruct((B, D, NPS), jnp.float32),
    )(gathered, new_xyz_embed)
    return out2.reshape(B, D, NP, NS)


def kernel(xyz, xyz_embed, new_xyz, new_xyz_embed, features):
    NP = new_xyz.shape[1]
    D = xyz_embed.shape[2] + features.shape[1]
    table = _build_table(xyz_embed, features)
    idx_flat = _ball_query_sc(xyz, new_xyz)
    gathered = _gather_sc(table, idx_flat)
    return _untranspose(gathered, new_xyz_embed, NP, D)


# speedup vs baseline: 10.9766x; 1.6552x over previous
import jax
import jax.numpy as jnp
from jax import lax
from jax.experimental import pallas as pl
from jax.experimental.pallas import tpu as pltpu
from jax.experimental.pallas import tpu_sc as plsc

_RADIUS = 0.2
_NSAMPLE = 32


def _build_table(xyz_embed, features):
    B, N, EMB = xyz_embed.shape
    C = features.shape[1]
    D = EMB + C
    DP = 384
    TN = 512
    n_blk = N // TN

    def body(emb_ref, feat_ref, out_ref):
        pad = jnp.zeros((TN, DP - D), jnp.float32)
        out_ref[...] = jnp.concatenate([emb_ref[0], feat_ref[0].T, pad], axis=1)

    return pl.pallas_call(
        body,
        grid=(B, n_blk),
        in_specs=[
            pl.BlockSpec((1, TN, EMB), lambda b, i: (b, i, 0)),
            pl.BlockSpec((1, C, TN), lambda b, i: (b, 0, i)),
        ],
        out_specs=pl.BlockSpec((TN, DP), lambda b, i: (b * n_blk + i, 0)),
        out_shape=jax.ShapeDtypeStruct((B * N, DP), jnp.float32),
    )(xyz_embed, features)


def _ball_query_sc(xyz, new_xyz):
    B, N, _ = xyz.shape
    NP = new_xyz.shape[1]
    NS = _NSAMPLE
    r2 = _RADIUS * _RADIUS

    info = plsc.get_sparse_core_info()
    NC, NSUB, L = info.num_cores, info.num_subcores, info.num_lanes
    NW = NC * NSUB
    CPW = (B * NP) // NW
    TOT = B * NP * NS
    n_chunks = N // L

    mesh = plsc.VectorSubcoreMesh(core_axis_name="c", subcore_axis_name="s")

    def body(xyz_hbm, new_hbm, out_hbm, xyz_v, new_v, idxbuf, outbuf):
        cid = lax.axis_index("c")
        sid = lax.axis_index("s")
        wid = sid * NC + cid
        g0 = wid * CPW
        b = g0 // NP
        p0 = g0 % NP
        pltpu.sync_copy(xyz_hbm.at[b], xyz_v)
        pltpu.sync_copy(new_hbm.at[b, pl.ds(p0 * 3, CPW * 3)], new_v)
        bN = b * N
        lanes = lax.broadcasted_iota(jnp.int32, (L,), 0)
        zeros = jnp.zeros((L,), jnp.int32)

        def per_centroid(k, carry):
            qbase = zeros + k * 3
            qx = plsc.load_gather(new_v, [qbase])
            qy = plsc.load_gather(new_v, [qbase + 1])
            qz = plsc.load_gather(new_v, [qbase + 2])
            idxbuf[pl.ds(0, L)] = jnp.full((L,), bN, jnp.int32)

            def cond(jc):
                j, cnt = jc
                return jnp.logical_and(j < n_chunks, cnt < NS)

            def step(jc):
                j, cnt = jc
                nidx = lanes + j * L
                nidx3 = nidx * 3
                px = plsc.load_gather(xyz_v, [nidx3])
                py = plsc.load_gather(xyz_v, [nidx3 + 1])
                pz = plsc.load_gather(xyz_v, [nidx3 + 2])
                dx = px - qx
                dy = py - qy
                dz = pz - qz
                d2 = dx * dx + dy * dy + dz * dz
                m = d2 <= r2
                plsc.store_compressed(idxbuf.at[pl.ds(cnt, L)], nidx + bN, mask=m)
                return j + 1, cnt + jnp.sum(m.astype(jnp.int32))

            _, total = lax.while_loop(cond, step, (jnp.int32(0), jnp.int32(0)))
            v0 = idxbuf[pl.ds(0, L)]
            v1 = idxbuf[pl.ds(L, L)]
            fvec = jnp.full((L,), v0[0], jnp.int32)
            kcol = zeros + k
            plsc.store_scatter(outbuf, [lanes, kcol],
                               jnp.where(lanes < total, v0, fvec))
            plsc.store_scatter(outbuf, [lanes + L, kcol],
                               jnp.where(lanes + L < total, v1, fvec))
            return carry

        lax.fori_loop(0, CPW, per_centroid, 0)
        pltpu.sync_copy(outbuf, out_hbm.at[pl.ds(b * NS, NS), pl.ds(p0, CPW)])

    bq = pl.kernel(
        body,
        out_type=jax.ShapeDtypeStruct((B * NS, NP), jnp.int32),
        mesh=mesh,
        compiler_params=pltpu.CompilerParams(needs_layout_passes=False),
        scratch_types=[
            pltpu.VMEM((N * 3,), jnp.float32),
            pltpu.VMEM((CPW * 3,), jnp.float32),
            pltpu.VMEM((4 * L,), jnp.int32),
            pltpu.VMEM((NS, CPW), jnp.int32),
        ],
    )
    return bq(xyz.reshape(B, N * 3), new_xyz.reshape(B, NP * 3)).reshape(TOT)


def _gather_sc(table, idx_flat):
    TOT = idx_flat.shape[0]
    D = table.shape[1]

    info = plsc.get_sparse_core_info()
    NC, NSUB = info.num_cores, info.num_subcores
    NW = NC * NSUB
    PW = TOT // NW
    CH = 128
    n_chunks = PW // CH

    mesh = plsc.VectorSubcoreMesh(core_axis_name="c", subcore_axis_name="s")

    def body(table_hbm, idx_hbm, out_hbm, idx_v, rows, sem):
        cid = lax.axis_index("c")
        sid = lax.axis_index("s")
        wid = sid * NC + cid
        base = wid * PW
        pltpu.sync_copy(idx_hbm.at[pl.ds(base, PW)], idx_v)

        def chunk(i, carry):
            off = i * CH
            pltpu.async_copy(table_hbm.at[idx_v.at[pl.ds(off, CH)]], rows, sem).wait()
            pltpu.sync_copy(rows, out_hbm.at[pl.ds(base + off, CH)])
            return carry

        lax.fori_loop(0, n_chunks, chunk, 0)

    g = pl.kernel(
        body,
        out_type=jax.ShapeDtypeStruct((TOT, D), jnp.float32),
        mesh=mesh,
        scratch_types=[
            pltpu.VMEM((PW,), jnp.int32),
            pltpu.VMEM((CH, D), jnp.float32),
            pltpu.SemaphoreType.DMA,
        ],
    )
    return g(table, idx_flat)


def _untranspose(gathered, new_xyz_embed, NP, D):
    TOT, DP = gathered.shape
    B, _, EMB = new_xyz_embed.shape
    NS = _NSAMPLE
    PT = 128
    n_blk = NP // PT
    g3 = gathered.reshape(B * NS, NP, DP)

    def body(g_ref, emb_ref, out_ref):
        et = emb_ref[0].T
        for s in range(NS):
            gt = g_ref[s, :, :D].T
            out_ref[0, :EMB, s, :] = gt[:EMB, :] - et
            out_ref[0, EMB:, s, :] = gt[EMB:, :]

    out3 = pl.pallas_call(
        body,
        grid=(B, n_blk),
        in_specs=[
            pl.BlockSpec((NS, PT, DP), lambda b, i: (b, i, 0)),
            pl.BlockSpec((1, PT, EMB), lambda b, i: (b, i, 0)),
        ],
        out_specs=pl.BlockSpec((1, D, NS, PT), lambda b, i: (b, 0, 0, i)),
        out_shape=jax.ShapeDtypeStruct((B, D, NS, NP), jnp.float32),
    )(g3, new_xyz_embed)
    return jnp.swapaxes(out3, 2, 3)


def kernel(xyz, xyz_embed, new_xyz, new_xyz_embed, features):
    NP = new_xyz.shape[1]
    D = xyz_embed.shape[2] + features.shape[1]
    table = _build_table(xyz_embed, features)
    idx_flat = _ball_query_sc(xyz, new_xyz)
    gathered = _gather_sc(table, idx_flat)
    return _untranspose(gathered, new_xyz_embed, NP, D)


# speedup vs baseline: 13.6463x; 1.2432x over previous
import jax
import jax.numpy as jnp
from jax import lax
from jax.experimental import pallas as pl
from jax.experimental.pallas import tpu as pltpu
from jax.experimental.pallas import tpu_sc as plsc

_RADIUS = 0.2
_NSAMPLE = 32


def _build_table(xyz_embed, features):
    B, N, EMB = xyz_embed.shape
    C = features.shape[1]
    D = EMB + C
    DP = 384
    TN = 512
    n_blk = N // TN

    def body(emb_ref, feat_ref, out_ref):
        pad = jnp.zeros((TN, DP - D), jnp.float32)
        out_ref[...] = jnp.concatenate([emb_ref[0], feat_ref[0].T, pad], axis=1)

    return pl.pallas_call(
        body,
        grid=(B, n_blk),
        in_specs=[
            pl.BlockSpec((1, TN, EMB), lambda b, i: (b, i, 0)),
            pl.BlockSpec((1, C, TN), lambda b, i: (b, 0, i)),
        ],
        out_specs=pl.BlockSpec((TN, DP), lambda b, i: (b * n_blk + i, 0)),
        out_shape=jax.ShapeDtypeStruct((B * N, DP), jnp.float32),
    )(xyz_embed, features)


def _ball_query_sc(xyz, new_xyz):
    B, N, _ = xyz.shape
    NP = new_xyz.shape[1]
    NS = _NSAMPLE
    r2 = _RADIUS * _RADIUS

    info = plsc.get_sparse_core_info()
    NC, NSUB, L = info.num_cores, info.num_subcores, info.num_lanes
    NW = NC * NSUB
    CPW = (B * NP) // NW
    TOT = B * NP * NS
    n_chunks = N // L

    mesh = plsc.VectorSubcoreMesh(core_axis_name="c", subcore_axis_name="s")

    def body(xyz_hbm, new_hbm, out_hbm, xyz_v, new_v, idxbuf, outbuf):
        cid = lax.axis_index("c")
        sid = lax.axis_index("s")
        wid = sid * NC + cid
        g0 = wid * CPW
        b = g0 // NP
        p0 = g0 % NP
        pltpu.sync_copy(xyz_hbm.at[b], xyz_v)
        pltpu.sync_copy(new_hbm.at[b, pl.ds(p0 * 3, CPW * 3)], new_v)
        bN = b * N
        lanes = lax.broadcasted_iota(jnp.int32, (L,), 0)
        zeros = jnp.zeros((L,), jnp.int32)

        lanes3 = lanes * 3

        def per_centroid(k, carry):
            qbase = zeros + k * 3
            qx = plsc.load_gather(new_v, [qbase])
            qy = plsc.load_gather(new_v, [qbase + 1])
            qz = plsc.load_gather(new_v, [qbase + 2])
            idxbuf[pl.ds(0, L)] = jnp.full((L,), bN, jnp.int32)

            def cond(jc):
                j, cnt = jc
                return jnp.logical_and(j < n_chunks, cnt < NS)

            def one(j, cnt):
                base3 = lanes3 + j * (3 * L)
                px = plsc.load_gather(xyz_v, [base3])
                py = plsc.load_gather(xyz_v, [base3 + 1])
                pz = plsc.load_gather(xyz_v, [base3 + 2])
                dx = px - qx
                dy = py - qy
                dz = pz - qz
                d2 = dx * dx + dy * dy + dz * dz
                m = d2 <= r2
                plsc.store_compressed(idxbuf.at[pl.ds(cnt, L)],
                                      lanes + (j * L + bN), mask=m)
                return cnt + plsc.all_reduce_population_count(m)[0]

            def step(jc):
                j, cnt = jc
                cnt = one(j, cnt)
                cnt = one(j + 1, cnt)
                return j + 2, cnt

            _, total = lax.while_loop(cond, step, (jnp.int32(0), jnp.int32(0)))
            v0 = idxbuf[pl.ds(0, L)]
            v1 = idxbuf[pl.ds(L, L)]
            fvec = jnp.full((L,), v0[0], jnp.int32)
            kcol = zeros + k
            plsc.store_scatter(outbuf, [lanes, kcol],
                               jnp.where(lanes < total, v0, fvec))
            plsc.store_scatter(outbuf, [lanes + L, kcol],
                               jnp.where(lanes + L < total, v1, fvec))
            return carry

        lax.fori_loop(0, CPW, per_centroid, 0)
        pltpu.sync_copy(outbuf, out_hbm.at[pl.ds(b * NS, NS), pl.ds(p0, CPW)])

    bq = pl.kernel(
        body,
        out_type=jax.ShapeDtypeStruct((B * NS, NP), jnp.int32),
        mesh=mesh,
        compiler_params=pltpu.CompilerParams(needs_layout_passes=False),
        scratch_types=[
            pltpu.VMEM((N * 3,), jnp.float32),
            pltpu.VMEM((CPW * 3,), jnp.float32),
            pltpu.VMEM((4 * L,), jnp.int32),
            pltpu.VMEM((NS, CPW), jnp.int32),
        ],
    )
    return bq(xyz.reshape(B, N * 3), new_xyz.reshape(B, NP * 3)).reshape(TOT)


def _gather_sc(table, idx_flat):
    TOT = idx_flat.shape[0]
    D = table.shape[1]

    info = plsc.get_sparse_core_info()
    NC, NSUB = info.num_cores, info.num_subcores
    NW = NC * NSUB
    PW = TOT // NW
    CH = 128
    n_chunks = PW // CH

    mesh = plsc.VectorSubcoreMesh(core_axis_name="c", subcore_axis_name="s")

    def body(table_hbm, idx_hbm, out_hbm, idx_v, rows, sem):
        cid = lax.axis_index("c")
        sid = lax.axis_index("s")
        wid = sid * NC + cid
        base = wid * PW
        pltpu.sync_copy(idx_hbm.at[pl.ds(base, PW)], idx_v)

        def chunk(i, carry):
            off = i * CH
            pltpu.async_copy(table_hbm.at[idx_v.at[pl.ds(off, CH)]], rows, sem).wait()
            pltpu.sync_copy(rows, out_hbm.at[pl.ds(base + off, CH)])
            return carry

        lax.fori_loop(0, n_chunks, chunk, 0)

    g = pl.kernel(
        body,
        out_type=jax.ShapeDtypeStruct((TOT, D), jnp.float32),
        mesh=mesh,
        scratch_types=[
            pltpu.VMEM((PW,), jnp.int32),
            pltpu.VMEM((CH, D), jnp.float32),
            pltpu.SemaphoreType.DMA,
        ],
    )
    return g(table, idx_flat)


def _untranspose(gathered, new_xyz_embed, NP, D):
    TOT, DP = gathered.shape
    B, _, EMB = new_xyz_embed.shape
    NS = _NSAMPLE
    PT = 128
    n_blk = NP // PT
    g3 = gathered.reshape(B * NS, NP, DP)

    def body(g_ref, emb_ref, out_ref):
        et = emb_ref[0].T
        for s in range(NS):
            gt = g_ref[s, :, :D].T
            out_ref[0, :EMB, s, :] = gt[:EMB, :] - et
            out_ref[0, EMB:, s, :] = gt[EMB:, :]

    out3 = pl.pallas_call(
        body,
        grid=(B, n_blk),
        in_specs=[
            pl.BlockSpec((NS, PT, DP), lambda b, i: (b, i, 0)),
            pl.BlockSpec((1, PT, EMB), lambda b, i: (b, i, 0)),
        ],
        out_specs=pl.BlockSpec((1, D, NS, PT), lambda b, i: (b, 0, 0, i)),
        out_shape=jax.ShapeDtypeStruct((B, D, NS, NP), jnp.float32),
    )(g3, new_xyz_embed)
    return jnp.swapaxes(out3, 2, 3)


def kernel(xyz, xyz_embed, new_xyz, new_xyz_embed, features):
    NP = new_xyz.shape[1]
    D = xyz_embed.shape[2] + features.shape[1]
    table = _build_table(xyz_embed, features)
    idx_flat = _ball_query_sc(xyz, new_xyz)
    gathered = _gather_sc(table, idx_flat)
    return _untranspose(gathered, new_xyz_embed, NP, D)


# speedup vs baseline: 14.6665x; 1.0748x over previous
import jax
import jax.numpy as jnp
from jax import lax
from jax.experimental import pallas as pl
from jax.experimental.pallas import tpu as pltpu
from jax.experimental.pallas import tpu_sc as plsc

_RADIUS = 0.2
_NSAMPLE = 32


def _build_table(xyz_embed, features):
    B, N, EMB = xyz_embed.shape
    C = features.shape[1]
    D = EMB + C
    DP = 384
    TN = 512
    n_blk = N // TN

    def body(emb_ref, feat_ref, out_ref):
        pad = jnp.zeros((TN, DP - D), jnp.float32)
        out_ref[...] = jnp.concatenate([emb_ref[0], feat_ref[0].T, pad], axis=1)

    return pl.pallas_call(
        body,
        grid=(B, n_blk),
        in_specs=[
            pl.BlockSpec((1, TN, EMB), lambda b, i: (b, i, 0)),
            pl.BlockSpec((1, C, TN), lambda b, i: (b, 0, i)),
        ],
        out_specs=pl.BlockSpec((TN, DP), lambda b, i: (b * n_blk + i, 0)),
        out_shape=jax.ShapeDtypeStruct((B * N, DP), jnp.float32),
    )(xyz_embed, features)


def _ball_query_sc(xyz, new_xyz):
    B, N, _ = xyz.shape
    NP = new_xyz.shape[1]
    NS = _NSAMPLE
    r2 = _RADIUS * _RADIUS

    info = plsc.get_sparse_core_info()
    NC, NSUB, L = info.num_cores, info.num_subcores, info.num_lanes
    NW = NC * NSUB
    CPW = (B * NP) // NW
    TOT = B * NP * NS
    n_chunks = N // L

    mesh = plsc.VectorSubcoreMesh(core_axis_name="c", subcore_axis_name="s")

    def body(xyz_hbm, new_hbm, out_hbm, xyz_v, new_v, idxbuf, outbuf):
        cid = lax.axis_index("c")
        sid = lax.axis_index("s")
        wid = sid * NC + cid
        g0 = wid * CPW
        b = g0 // NP
        p0 = g0 % NP
        pltpu.sync_copy(xyz_hbm.at[b], xyz_v)
        pltpu.sync_copy(new_hbm.at[b, pl.ds(p0 * 3, CPW * 3)], new_v)
        bN = b * N
        lanes = lax.broadcasted_iota(jnp.int32, (L,), 0)
        zeros = jnp.zeros((L,), jnp.int32)

        lanes3 = lanes * 3

        def per_centroid(k, carry):
            qbase = zeros + k * 3
            qx = plsc.load_gather(new_v, [qbase])
            qy = plsc.load_gather(new_v, [qbase + 1])
            qz = plsc.load_gather(new_v, [qbase + 2])
            idxbuf[pl.ds(0, L)] = jnp.full((L,), bN, jnp.int32)

            def cond(jc):
                j, cnt = jc
                return jnp.logical_and(j < n_chunks, cnt < NS)

            def one(j, cnt):
                base3 = lanes3 + j * (3 * L)
                px = plsc.load_gather(xyz_v, [base3])
                py = plsc.load_gather(xyz_v, [base3 + 1])
                pz = plsc.load_gather(xyz_v, [base3 + 2])
                dx = px - qx
                dy = py - qy
                dz = pz - qz
                d2 = dx * dx + dy * dy + dz * dz
                m = d2 <= r2
                plsc.store_compressed(idxbuf.at[pl.ds(cnt, L)],
                                      lanes + (j * L + bN), mask=m)
                return cnt + plsc.all_reduce_population_count(m)[0]

            def step(jc):
                j, cnt = jc
                cnt = one(j, cnt)
                cnt = one(j + 1, cnt)
                cnt = one(j + 2, cnt)
                cnt = one(j + 3, cnt)
                return j + 4, cnt

            _, total = lax.while_loop(cond, step, (jnp.int32(0), jnp.int32(0)))
            v0 = idxbuf[pl.ds(0, L)]
            v1 = idxbuf[pl.ds(L, L)]
            fvec = jnp.full((L,), v0[0], jnp.int32)
            kcol = zeros + k
            plsc.store_scatter(outbuf, [lanes, kcol],
                               jnp.where(lanes < total, v0, fvec))
            plsc.store_scatter(outbuf, [lanes + L, kcol],
                               jnp.where(lanes + L < total, v1, fvec))
            return carry

        lax.fori_loop(0, CPW, per_centroid, 0)
        pltpu.sync_copy(outbuf, out_hbm.at[pl.ds(b * NS, NS), pl.ds(p0, CPW)])

    bq = pl.kernel(
        body,
        out_type=jax.ShapeDtypeStruct((B * NS, NP), jnp.int32),
        mesh=mesh,
        compiler_params=pltpu.CompilerParams(needs_layout_passes=False),
        scratch_types=[
            pltpu.VMEM((N * 3,), jnp.float32),
            pltpu.VMEM((CPW * 3,), jnp.float32),
            pltpu.VMEM((8 * L,), jnp.int32),
            pltpu.VMEM((NS, CPW), jnp.int32),
        ],
    )
    return bq(xyz.reshape(B, N * 3), new_xyz.reshape(B, NP * 3)).reshape(TOT)


def _gather_sc(table, idx_flat):
    TOT = idx_flat.shape[0]
    D = table.shape[1]

    info = plsc.get_sparse_core_info()
    NC, NSUB = info.num_cores, info.num_subcores
    NW = NC * NSUB
    PW = TOT // NW
    CH = 128
    n_chunks = PW // CH

    mesh = plsc.VectorSubcoreMesh(core_axis_name="c", subcore_axis_name="s")

    def body(table_hbm, idx_hbm, out_hbm, idx_v, rows, sem):
        cid = lax.axis_index("c")
        sid = lax.axis_index("s")
        wid = sid * NC + cid
        base = wid * PW
        pltpu.sync_copy(idx_hbm.at[pl.ds(base, PW)], idx_v)

        def chunk(i, carry):
            off = i * CH
            pltpu.async_copy(table_hbm.at[idx_v.at[pl.ds(off, CH)]], rows, sem).wait()
            pltpu.sync_copy(rows, out_hbm.at[pl.ds(base + off, CH)])
            return carry

        lax.fori_loop(0, n_chunks, chunk, 0)

    g = pl.kernel(
        body,
        out_type=jax.ShapeDtypeStruct((TOT, D), jnp.float32),
        mesh=mesh,
        scratch_types=[
            pltpu.VMEM((PW,), jnp.int32),
            pltpu.VMEM((CH, D), jnp.float32),
            pltpu.SemaphoreType.DMA,
        ],
    )
    return g(table, idx_flat)


def _untranspose(gathered, new_xyz_embed, NP, D):
    TOT, DP = gathered.shape
    B, _, EMB = new_xyz_embed.shape
    NS = _NSAMPLE
    PT = 128
    n_blk = NP // PT
    g3 = gathered.reshape(B * NS, NP, DP)

    def body(g_ref, emb_ref, out_ref):
        et = emb_ref[0].T
        for s in range(NS):
            gt = g_ref[s, :, :D].T
            out_ref[0, :EMB, s, :] = gt[:EMB, :] - et
            out_ref[0, EMB:, s, :] = gt[EMB:, :]

    out3 = pl.pallas_call(
        body,
        grid=(B, n_blk),
        in_specs=[
            pl.BlockSpec((NS, PT, DP), lambda b, i: (b, i, 0)),
            pl.BlockSpec((1, PT, EMB), lambda b, i: (b, i, 0)),
        ],
        out_specs=pl.BlockSpec((1, D, NS, PT), lambda b, i: (b, 0, 0, i)),
        out_shape=jax.ShapeDtypeStruct((B, D, NS, NP), jnp.float32),
    )(g3, new_xyz_embed)
    return jnp.swapaxes(out3, 2, 3)


def kernel(xyz, xyz_embed, new_xyz, new_xyz_embed, features):
    NP = new_xyz.shape[1]
    D = xyz_embed.shape[2] + features.shape[1]
    table = _build_table(xyz_embed, features)
    idx_flat = _ball_query_sc(xyz, new_xyz)
    gathered = _gather_sc(table, idx_flat)
    return _untranspose(gathered, new_xyz_embed, NP, D)


# speedup vs baseline: 14.9690x; 1.0206x over previous
import jax
import jax.numpy as jnp
from jax import lax
from jax.experimental import pallas as pl
from jax.experimental.pallas import tpu as pltpu
from jax.experimental.pallas import tpu_sc as plsc

_RADIUS = 0.2
_NSAMPLE = 32


def _build_table(xyz_embed, features):
    B, N, EMB = xyz_embed.shape
    C = features.shape[1]
    D = EMB + C
    DP = 384
    TN = 512
    n_blk = N // TN

    def body(emb_ref, feat_ref, out_ref):
        pad = jnp.zeros((TN, DP - D), jnp.float32)
        out_ref[...] = jnp.concatenate([emb_ref[0], feat_ref[0].T, pad], axis=1)

    return pl.pallas_call(
        body,
        grid=(B, n_blk),
        in_specs=[
            pl.BlockSpec((1, TN, EMB), lambda b, i: (b, i, 0)),
            pl.BlockSpec((1, C, TN), lambda b, i: (b, 0, i)),
        ],
        out_specs=pl.BlockSpec((TN, DP), lambda b, i: (b * n_blk + i, 0)),
        out_shape=jax.ShapeDtypeStruct((B * N, DP), jnp.float32),
    )(xyz_embed, features)


def _ball_query_sc(xyz, new_xyz):
    B, N, _ = xyz.shape
    NP = new_xyz.shape[1]
    NS = _NSAMPLE
    r2 = _RADIUS * _RADIUS

    info = plsc.get_sparse_core_info()
    NC, NSUB, L = info.num_cores, info.num_subcores, info.num_lanes
    NW = NC * NSUB
    CPW = (B * NP) // NW
    TOT = B * NP * NS
    n_chunks = N // L

    mesh = plsc.VectorSubcoreMesh(core_axis_name="c", subcore_axis_name="s")

    def body(xyz_hbm, new_hbm, out_hbm, xyz_v, new_v, idxbuf, outbuf):
        cid = lax.axis_index("c")
        sid = lax.axis_index("s")
        wid = sid * NC + cid
        g0 = wid * CPW
        b = g0 // NP
        p0 = g0 % NP
        pltpu.sync_copy(xyz_hbm.at[b], xyz_v)
        pltpu.sync_copy(new_hbm.at[b, pl.ds(p0 * 3, CPW * 3)], new_v)
        bN = b * N
        lanes = lax.broadcasted_iota(jnp.int32, (L,), 0)
        zeros = jnp.zeros((L,), jnp.int32)

        lanes3 = lanes * 3

        def per_centroid(k, carry):
            qbase = zeros + k * 3
            qx = plsc.load_gather(new_v, [qbase])
            qy = plsc.load_gather(new_v, [qbase + 1])
            qz = plsc.load_gather(new_v, [qbase + 2])
            idxbuf[pl.ds(0, L)] = jnp.full((L,), bN, jnp.int32)

            def cond(jc):
                j, cnt = jc
                return jnp.logical_and(j < n_chunks, cnt < NS)

            def one(j, cnt):
                base3 = lanes3 + j * (3 * L)
                px = plsc.load_gather(xyz_v, [base3])
                py = plsc.load_gather(xyz_v, [base3 + 1])
                pz = plsc.load_gather(xyz_v, [base3 + 2])
                dx = px - qx
                dy = py - qy
                dz = pz - qz
                d2 = dx * dx + dy * dy + dz * dz
                m = d2 <= r2
                plsc.store_compressed(idxbuf.at[pl.ds(cnt, L)],
                                      lanes + (j * L + bN), mask=m)
                return cnt + plsc.all_reduce_population_count(m)[0]

            def step(jc):
                j, cnt = jc
                cnt = one(j, cnt)
                cnt = one(j + 1, cnt)
                cnt = one(j + 2, cnt)
                cnt = one(j + 3, cnt)
                return j + 4, cnt

            _, total = lax.while_loop(cond, step, (jnp.int32(0), jnp.int32(0)))
            v0 = idxbuf[pl.ds(0, L)]
            v1 = idxbuf[pl.ds(L, L)]
            fvec = jnp.full((L,), v0[0], jnp.int32)
            kcol = zeros + k
            plsc.store_scatter(outbuf, [lanes, kcol],
                               jnp.where(lanes < total, v0, fvec))
            plsc.store_scatter(outbuf, [lanes + L, kcol],
                               jnp.where(lanes + L < total, v1, fvec))
            return carry

        lax.fori_loop(0, CPW, per_centroid, 0)
        pltpu.sync_copy(outbuf, out_hbm.at[pl.ds(b * NS, NS), pl.ds(p0, CPW)])

    bq = pl.kernel(
        body,
        out_type=jax.ShapeDtypeStruct((B * NS, NP), jnp.int32),
        mesh=mesh,
        compiler_params=pltpu.CompilerParams(needs_layout_passes=False),
        scratch_types=[
            pltpu.VMEM((N * 3,), jnp.float32),
            pltpu.VMEM((CPW * 3,), jnp.float32),
            pltpu.VMEM((8 * L,), jnp.int32),
            pltpu.VMEM((NS, CPW), jnp.int32),
        ],
    )
    return bq(xyz.reshape(B, N * 3), new_xyz.reshape(B, NP * 3)).reshape(TOT)


def _gather_sc(table, idx_flat):
    TOT = idx_flat.shape[0]
    D = table.shape[1]

    info = plsc.get_sparse_core_info()
    NC, NSUB = info.num_cores, info.num_subcores
    NW = NC * NSUB
    PW = TOT // NW
    CH = 128
    n_chunks = PW // CH

    mesh = plsc.VectorSubcoreMesh(core_axis_name="c", subcore_axis_name="s")

    def body(table_hbm, idx_hbm, out_hbm, idx_v, rows0, rows1, gs0, gs1, ss0, ss1):
        cid = lax.axis_index("c")
        sid = lax.axis_index("s")
        wid = sid * NC + cid
        base = wid * PW
        pltpu.sync_copy(idx_hbm.at[pl.ds(base, PW)], idx_v)
        rows = (rows0, rows1)
        gs = (gs0, gs1)
        ss = (ss0, ss1)

        def gather_start(i, buf):
            pltpu.make_async_copy(
                table_hbm.at[idx_v.at[pl.ds(i * CH, CH)]], rows[buf], gs[buf]
            ).start()

        def gather_wait(buf):
            pltpu.make_async_copy(
                table_hbm.at[idx_v.at[pl.ds(0, CH)]], rows[buf], gs[buf]
            ).wait()

        def store_start(i, buf):
            pltpu.make_async_copy(
                rows[buf], out_hbm.at[pl.ds(base + i * CH, CH)], ss[buf]
            ).start()

        def store_wait(buf):
            pltpu.make_async_copy(
                rows[buf], out_hbm.at[pl.ds(base, CH)], ss[buf]
            ).wait()

        gather_start(0, 0)
        gather_start(1, 1)

        def loop(i2, carry):
            i = i2 * 2
            gather_wait(0)
            store_start(i, 0)
            gather_wait(1)
            store_start(i + 1, 1)

            @pl.when(i + 2 < n_chunks)
            def _():
                store_wait(0)
                gather_start(i + 2, 0)
                store_wait(1)
                gather_start(i + 3, 1)

            return carry

        lax.fori_loop(0, n_chunks // 2, loop, 0)
        store_wait(0)
        store_wait(1)

    g = pl.kernel(
        body,
        out_type=jax.ShapeDtypeStruct((TOT, D), jnp.float32),
        mesh=mesh,
        scratch_types=[
            pltpu.VMEM((PW,), jnp.int32),
            pltpu.VMEM((CH, D), jnp.float32),
            pltpu.VMEM((CH, D), jnp.float32),
            pltpu.SemaphoreType.DMA,
            pltpu.SemaphoreType.DMA,
            pltpu.SemaphoreType.DMA,
            pltpu.SemaphoreType.DMA,
        ],
    )
    return g(table, idx_flat)


def _untranspose(gathered, new_xyz_embed, NP, D):
    TOT, DP = gathered.shape
    B, _, EMB = new_xyz_embed.shape
    NS = _NSAMPLE
    PT = 128
    n_blk = NP // PT
    g3 = gathered.reshape(B * NS, NP, DP)

    def body(g_ref, emb_ref, out_ref):
        et = emb_ref[0].T
        for s in range(NS):
            gt = g_ref[s, :, :D].T
            out_ref[0, :EMB, s, :] = gt[:EMB, :] - et
            out_ref[0, EMB:, s, :] = gt[EMB:, :]

    out3 = pl.pallas_call(
        body,
        grid=(B, n_blk),
        in_specs=[
            pl.BlockSpec((NS, PT, DP), lambda b, i: (b, i, 0)),
            pl.BlockSpec((1, PT, EMB), lambda b, i: (b, i, 0)),
        ],
        out_specs=pl.BlockSpec((1, D, NS, PT), lambda b, i: (b, 0, 0, i)),
        out_shape=jax.ShapeDtypeStruct((B, D, NS, NP), jnp.float32),
    )(g3, new_xyz_embed)
    return jnp.swapaxes(out3, 2, 3)


def kernel(xyz, xyz_embed, new_xyz, new_xyz_embed, features):
    NP = new_xyz.shape[1]
    D = xyz_embed.shape[2] + features.shape[1]
    table = _build_table(xyz_embed, features)
    idx_flat = _ball_query_sc(xyz, new_xyz)
    gathered = _gather_sc(table, idx_flat)
    return _untranspose(gathered, new_xyz_embed, NP, D)


# speedup vs baseline: 18.1780x; 1.2144x over previous
import jax
import jax.numpy as jnp
from jax import lax
from jax.experimental import pallas as pl
from jax.experimental.pallas import tpu as pltpu
from jax.experimental.pallas import tpu_sc as plsc

_RADIUS = 0.2
_NSAMPLE = 32


def _build_table(xyz_embed, features):
    B, N, EMB = xyz_embed.shape
    C = features.shape[1]
    D = EMB + C
    DP = 384
    TN = 512
    n_blk = N // TN

    def body(emb_ref, feat_ref, out_ref):
        pad = jnp.zeros((TN, DP - D), jnp.float32)
        out_ref[...] = jnp.concatenate([emb_ref[0], feat_ref[0].T, pad], axis=1)

    return pl.pallas_call(
        body,
        grid=(B, n_blk),
        in_specs=[
            pl.BlockSpec((1, TN, EMB), lambda b, i: (b, i, 0)),
            pl.BlockSpec((1, C, TN), lambda b, i: (b, 0, i)),
        ],
        out_specs=pl.BlockSpec((TN, DP), lambda b, i: (b * n_blk + i, 0)),
        out_shape=jax.ShapeDtypeStruct((B * N, DP), jnp.float32),
    )(xyz_embed, features)


def _bq_gather_sc(xyz, new_xyz, table):
    B, N, _ = xyz.shape
    NP = new_xyz.shape[1]
    NS = _NSAMPLE
    DP = table.shape[1]
    r2 = _RADIUS * _RADIUS

    info = plsc.get_sparse_core_info()
    NC, NSUB, L = info.num_cores, info.num_subcores, info.num_lanes
    NW = NC * NSUB
    CPW = (B * NP) // NW
    n_chunks = N // L
    NB = 4

    mesh = plsc.VectorSubcoreMesh(core_axis_name="c", subcore_axis_name="s")

    def body(xyz_hbm, new_hbm, table_hbm, out_hbm, xyz_v, new_v, idxbuf,
             idxstage, rowbufs, gsems, ssems):
        cid = lax.axis_index("c")
        sid = lax.axis_index("s")
        wid = sid * NC + cid
        g0 = wid * CPW
        b = g0 // NP
        p0 = g0 % NP
        pltpu.sync_copy(xyz_hbm.at[b], xyz_v)
        pltpu.sync_copy(new_hbm.at[b, pl.ds(p0 * 3, CPW * 3)], new_v)
        bN = b * N
        bNS = b * NS
        lanes = lax.broadcasted_iota(jnp.int32, (L,), 0)
        zeros = jnp.zeros((L,), jnp.int32)
        lanes3 = lanes * 3

        def gather_start(buf):
            pltpu.make_async_copy(
                table_hbm.at[idxstage.at[buf]], rowbufs[buf], gsems[buf]
            ).start()

        def gather_wait(buf):
            pltpu.make_async_copy(
                table_hbm.at[idxstage.at[0]], rowbufs[buf], gsems[buf]
            ).wait()

        def store_start(buf, p):
            pltpu.make_async_copy(
                rowbufs[buf], out_hbm.at[pl.ds(bNS, NS), p], ssems[buf]
            ).start()

        def store_wait(buf):
            pltpu.make_async_copy(
                rowbufs[buf], out_hbm.at[pl.ds(bNS, NS), 0], ssems[buf]
            ).wait()

        def ball_query(k, buf):
            qbase = zeros + k * 3
            qx = plsc.load_gather(new_v, [qbase])
            qy = plsc.load_gather(new_v, [qbase + 1])
            qz = plsc.load_gather(new_v, [qbase + 2])
            idxbuf[pl.ds(0, L)] = jnp.full((L,), bN, jnp.int32)

            def cond(jc):
                j, cnt = jc
                return jnp.logical_and(j < n_chunks, cnt < NS)

            def one(j, cnt):
                base3 = lanes3 + j * (3 * L)
                px = plsc.load_gather(xyz_v, [base3])
                py = plsc.load_gather(xyz_v, [base3 + 1])
                pz = plsc.load_gather(xyz_v, [base3 + 2])
                dx = px - qx
                dy = py - qy
                dz = pz - qz
                d2 = dx * dx + dy * dy + dz * dz
                m = d2 <= r2
                plsc.store_compressed(idxbuf.at[pl.ds(cnt, L)],
                                      lanes + (j * L + bN), mask=m)
                return cnt + plsc.all_reduce_population_count(m)[0]

            def step(jc):
                j, cnt = jc
                cnt = one(j, cnt)
                cnt = one(j + 1, cnt)
                cnt = one(j + 2, cnt)
                cnt = one(j + 3, cnt)
                return j + 4, cnt

            _, total = lax.while_loop(cond, step, (jnp.int32(0), jnp.int32(0)))
            v0 = idxbuf[pl.ds(0, L)]
            v1 = idxbuf[pl.ds(L, L)]
            fvec = jnp.full((L,), v0[0], jnp.int32)
            idxstage[buf, pl.ds(0, L)] = jnp.where(lanes < total, v0, fvec)
            idxstage[buf, pl.ds(L, L)] = jnp.where(lanes + L < total, v1, fvec)

        def per_round(r, carry):
            k0 = r * NB
            for j in range(NB):
                k = k0 + j

                @pl.when(k >= NB)
                def _(j=j):
                    store_wait(j)

                ball_query(k, j)
                gather_start(j)

                pbuf = (j - 1) % NB

                @pl.when(k >= 1)
                def _(pbuf=pbuf, k=k):
                    gather_wait(pbuf)
                    store_start(pbuf, p0 + (k - 1))

            return carry

        lax.fori_loop(0, CPW // NB, per_round, 0)
        lbuf = (CPW - 1) % NB
        gather_wait(lbuf)
        store_start(lbuf, p0 + (CPW - 1))
        for buf in range(NB):
            store_wait(buf)

    fused = pl.kernel(
        body,
        out_type=jax.ShapeDtypeStruct((B * NS, NP, DP), jnp.float32),
        mesh=mesh,
        compiler_params=pltpu.CompilerParams(needs_layout_passes=False),
        scratch_types=[
            pltpu.VMEM((N * 3,), jnp.float32),
            pltpu.VMEM((CPW * 3,), jnp.float32),
            pltpu.VMEM((8 * L,), jnp.int32),
            pltpu.VMEM((NB, 2 * L), jnp.int32),
            [pltpu.VMEM((NS, DP), jnp.float32) for _ in range(NB)],
            [pltpu.SemaphoreType.DMA for _ in range(NB)],
            [pltpu.SemaphoreType.DMA for _ in range(NB)],
        ],
    )
    return fused(xyz.reshape(B, N * 3), new_xyz.reshape(B, NP * 3), table)


def _untranspose(gathered, new_xyz_embed, NP, D):
    TOT, DP = gathered.shape
    B, _, EMB = new_xyz_embed.shape
    NS = _NSAMPLE
    PT = 128
    n_blk = NP // PT
    g3 = gathered.reshape(B * NS, NP, DP)

    def body(g_ref, emb_ref, out_ref):
        et = emb_ref[0].T
        for s in range(NS):
            gt = g_ref[s, :, :D].T
            out_ref[0, :EMB, s, :] = gt[:EMB, :] - et
            out_ref[0, EMB:, s, :] = gt[EMB:, :]

    out3 = pl.pallas_call(
        body,
        grid=(B, n_blk),
        in_specs=[
            pl.BlockSpec((NS, PT, DP), lambda b, i: (b, i, 0)),
            pl.BlockSpec((1, PT, EMB), lambda b, i: (b, i, 0)),
        ],
        out_specs=pl.BlockSpec((1, D, NS, PT), lambda b, i: (b, 0, 0, i)),
        out_shape=jax.ShapeDtypeStruct((B, D, NS, NP), jnp.float32),
    )(g3, new_xyz_embed)
    return jnp.swapaxes(out3, 2, 3)


def kernel(xyz, xyz_embed, new_xyz, new_xyz_embed, features):
    NP = new_xyz.shape[1]
    D = xyz_embed.shape[2] + features.shape[1]
    table = _build_table(xyz_embed, features)
    g3 = _bq_gather_sc(xyz, new_xyz, table)
    gathered = g3.reshape(g3.shape[0] * NP, g3.shape[2])
    return _untranspose(gathered, new_xyz_embed, NP, D)


# speedup vs baseline: 20.0130x; 1.1009x over previous
import jax
import jax.numpy as jnp
from jax import lax
from jax.experimental import pallas as pl
from jax.experimental.pallas import tpu as pltpu
from jax.experimental.pallas import tpu_sc as plsc

_RADIUS = 0.2
_NSAMPLE = 32


def _build_table(xyz_embed, features):
    B, N, EMB = xyz_embed.shape
    C = features.shape[1]
    D = EMB + C
    DP = 384
    TN = 512
    n_blk = N // TN

    def body(emb_ref, feat_ref, out_ref):
        pad = jnp.zeros((TN, DP - D), jnp.float32)
        out_ref[...] = jnp.concatenate([emb_ref[0], feat_ref[0].T, pad], axis=1)

    return pl.pallas_call(
        body,
        grid=(B, n_blk),
        in_specs=[
            pl.BlockSpec((1, TN, EMB), lambda b, i: (b, i, 0)),
            pl.BlockSpec((1, C, TN), lambda b, i: (b, 0, i)),
        ],
        out_specs=pl.BlockSpec((TN, DP), lambda b, i: (b * n_blk + i, 0)),
        out_shape=jax.ShapeDtypeStruct((B * N, DP), jnp.float32),
    )(xyz_embed, features)


def _bq_gather_sc(xyz, new_xyz, table):
    B, N, _ = xyz.shape
    NP = new_xyz.shape[1]
    NS = _NSAMPLE
    DP = table.shape[1]
    r2 = _RADIUS * _RADIUS

    info = plsc.get_sparse_core_info()
    NC, NSUB, L = info.num_cores, info.num_subcores, info.num_lanes
    NW = NC * NSUB
    CPW = (B * NP) // NW
    n_chunks = N // L
    NB = 4

    mesh = plsc.VectorSubcoreMesh(core_axis_name="c", subcore_axis_name="s")

    def body(xyz_hbm, new_hbm, table_hbm, out_hbm, xyz_v, new_v, idxbuf,
             idxstage, rowbufs, gsems, ssems):
        cid = lax.axis_index("c")
        sid = lax.axis_index("s")
        wid = sid * NC + cid
        g0 = wid * CPW
        b = g0 // NP
        p0 = g0 % NP
        pltpu.sync_copy(xyz_hbm.at[b], xyz_v)
        pltpu.sync_copy(new_hbm.at[b, pl.ds(p0 * 3, CPW * 3)], new_v)
        bN = b * N
        bNS = b * NS
        lanes = lax.broadcasted_iota(jnp.int32, (L,), 0)
        zeros = jnp.zeros((L,), jnp.int32)
        lanes3 = lanes * 3

        def gather_start(buf):
            pltpu.make_async_copy(
                table_hbm.at[idxstage.at[buf]], rowbufs[buf], gsems[buf]
            ).start()

        def gather_wait(buf):
            pltpu.make_async_copy(
                table_hbm.at[idxstage.at[0]], rowbufs[buf], gsems[buf]
            ).wait()

        def store_start(buf, p):
            pltpu.make_async_copy(
                rowbufs[buf], out_hbm.at[pl.ds(bNS, NS), p], ssems[buf]
            ).start()

        def store_wait(buf):
            pltpu.make_async_copy(
                rowbufs[buf], out_hbm.at[pl.ds(bNS, NS), 0], ssems[buf]
            ).wait()

        def ball_query(k, buf):
            qbase = zeros + k * 3
            qx = plsc.load_gather(new_v, [qbase])
            qy = plsc.load_gather(new_v, [qbase + 1])
            qz = plsc.load_gather(new_v, [qbase + 2])
            idxbuf[pl.ds(0, L)] = jnp.full((L,), bN, jnp.int32)

            def cond(jc):
                j, cnt = jc
                return jnp.logical_and(j < n_chunks, cnt < NS)

            def one(j, cnt):
                base3 = lanes3 + j * (3 * L)
                px = plsc.load_gather(xyz_v, [base3])
                py = plsc.load_gather(xyz_v, [base3 + 1])
                pz = plsc.load_gather(xyz_v, [base3 + 2])
                dx = px - qx
                dy = py - qy
                dz = pz - qz
                d2 = dx * dx + dy * dy + dz * dz
                m = d2 <= r2
                plsc.store_compressed(idxbuf.at[pl.ds(cnt, L)],
                                      lanes + (j * L + bN), mask=m)
                return cnt + plsc.all_reduce_population_count(m)[0]

            def step(jc):
                j, cnt = jc
                cnt = one(j, cnt)
                cnt = one(j + 1, cnt)
                cnt = one(j + 2, cnt)
                cnt = one(j + 3, cnt)
                return j + 4, cnt

            _, total = lax.while_loop(cond, step, (jnp.int32(0), jnp.int32(0)))
            v0 = idxbuf[pl.ds(0, L)]
            v1 = idxbuf[pl.ds(L, L)]
            fvec = jnp.full((L,), v0[0], jnp.int32)
            idxstage[buf, pl.ds(0, L)] = jnp.where(lanes < total, v0, fvec)
            idxstage[buf, pl.ds(L, L)] = jnp.where(lanes + L < total, v1, fvec)

        def per_round(r, carry):
            k0 = r * NB
            for j in range(NB):
                k = k0 + j

                @pl.when(k >= NB)
                def _(j=j):
                    store_wait(j)

                ball_query(k, j)
                gather_start(j)

                pbuf = (j - 1) % NB

                @pl.when(k >= 1)
                def _(pbuf=pbuf, k=k):
                    gather_wait(pbuf)
                    store_start(pbuf, p0 + (k - 1))

            return carry

        lax.fori_loop(0, CPW // NB, per_round, 0)
        lbuf = (CPW - 1) % NB
        gather_wait(lbuf)
        store_start(lbuf, p0 + (CPW - 1))
        for buf in range(NB):
            store_wait(buf)

    fused = pl.kernel(
        body,
        out_type=jax.ShapeDtypeStruct((B * NS, NP, DP), jnp.float32),
        mesh=mesh,
        compiler_params=pltpu.CompilerParams(needs_layout_passes=False),
        scratch_types=[
            pltpu.VMEM((N * 3,), jnp.float32),
            pltpu.VMEM((CPW * 3,), jnp.float32),
            pltpu.VMEM((8 * L,), jnp.int32),
            pltpu.VMEM((NB, 2 * L), jnp.int32),
            [pltpu.VMEM((NS, DP), jnp.float32) for _ in range(NB)],
            [pltpu.SemaphoreType.DMA for _ in range(NB)],
            [pltpu.SemaphoreType.DMA for _ in range(NB)],
        ],
    )
    return fused(xyz.reshape(B, N * 3), new_xyz.reshape(B, NP * 3), table)


def _untranspose(gathered, new_xyz_embed, NP, D):
    TOT, DP = gathered.shape
    B, _, EMB = new_xyz_embed.shape
    NS = _NSAMPLE
    PT = 256
    n_blk = NP // PT
    g3 = gathered.reshape(B * NS, NP, DP)

    def body(g_ref, emb_ref, out_ref):
        et = emb_ref[0].T
        for s in range(NS):
            gt = g_ref[s, :, :D].T
            out_ref[0, :EMB, s, :] = gt[:EMB, :] - et
            out_ref[0, EMB:, s, :] = gt[EMB:, :]

    out3 = pl.pallas_call(
        body,
        grid=(B, n_blk),
        in_specs=[
            pl.BlockSpec((NS, PT, DP), lambda b, i: (b, i, 0)),
            pl.BlockSpec((1, PT, EMB), lambda b, i: (b, i, 0)),
        ],
        out_specs=pl.BlockSpec((1, D, NS, PT), lambda b, i: (b, 0, 0, i)),
        out_shape=jax.ShapeDtypeStruct((B, D, NS, NP), jnp.float32),
    )(g3, new_xyz_embed)
    return jnp.swapaxes(out3, 2, 3)


def kernel(xyz, xyz_embed, new_xyz, new_xyz_embed, features):
    NP = new_xyz.shape[1]
    D = xyz_embed.shape[2] + features.shape[1]
    table = _build_table(xyz_embed, features)
    g3 = _bq_gather_sc(xyz, new_xyz, table)
    gathered = g3.reshape(g3.shape[0] * NP, g3.shape[2])
    return _untranspose(gathered, new_xyz_embed, NP, D)


# speedup vs baseline: 20.6932x; 1.0340x over previous
import jax
import jax.numpy as jnp
from jax import lax
from jax.experimental import pallas as pl
from jax.experimental.pallas import tpu as pltpu
from jax.experimental.pallas import tpu_sc as plsc

_RADIUS = 0.2
_NSAMPLE = 32


def _build_table(xyz_embed, features, xyz):
    B, N, EMB = xyz_embed.shape
    C = features.shape[1]
    D = EMB + C
    DP = 384
    TN = 512
    n_blk = N // TN

    def body(emb_ref, feat_ref, xyz_ref, out_ref, pl_ref):
        pad = jnp.zeros((TN, DP - D), jnp.float32)
        out_ref[...] = jnp.concatenate([emb_ref[0], feat_ref[0].T, pad], axis=1)
        pl_ref[0] = xyz_ref[0].T

    return pl.pallas_call(
        body,
        grid=(B, n_blk),
        in_specs=[
            pl.BlockSpec((1, TN, EMB), lambda b, i: (b, i, 0)),
            pl.BlockSpec((1, C, TN), lambda b, i: (b, 0, i)),
            pl.BlockSpec((1, TN, 3), lambda b, i: (b, i, 0)),
        ],
        out_specs=[
            pl.BlockSpec((TN, DP), lambda b, i: (b * n_blk + i, 0)),
            pl.BlockSpec((1, 3, TN), lambda b, i: (b, 0, i)),
        ],
        out_shape=[
            jax.ShapeDtypeStruct((B * N, DP), jnp.float32),
            jax.ShapeDtypeStruct((B, 3, N), jnp.float32),
        ],
    )(xyz_embed, features, xyz)


def _bq_gather_sc(xyz, new_xyz, table):
    B, _, N = xyz.shape
    NP = new_xyz.shape[1]
    NS = _NSAMPLE
    DP = table.shape[1]
    r2 = _RADIUS * _RADIUS

    info = plsc.get_sparse_core_info()
    NC, NSUB, L = info.num_cores, info.num_subcores, info.num_lanes
    NW = NC * NSUB
    CPW = (B * NP) // NW
    n_chunks = N // L
    NB = 4

    mesh = plsc.VectorSubcoreMesh(core_axis_name="c", subcore_axis_name="s")

    def body(xyz_hbm, new_hbm, table_hbm, out_hbm, xyz_v, new_v, idxbuf,
             idxstage, rowbufs, gsems, ssems):
        cid = lax.axis_index("c")
        sid = lax.axis_index("s")
        wid = sid * NC + cid
        g0 = wid * CPW
        b = g0 // NP
        p0 = g0 % NP
        pltpu.sync_copy(xyz_hbm.at[b], xyz_v)
        pltpu.sync_copy(new_hbm.at[b, pl.ds(p0 * 3, CPW * 3)], new_v)
        bN = b * N
        bNS = b * NS
        lanes = lax.broadcasted_iota(jnp.int32, (L,), 0)
        zeros = jnp.zeros((L,), jnp.int32)

        def gather_start(buf):
            pltpu.make_async_copy(
                table_hbm.at[idxstage.at[buf]], rowbufs[buf], gsems[buf]
            ).start()

        def gather_wait(buf):
            pltpu.make_async_copy(
                table_hbm.at[idxstage.at[0]], rowbufs[buf], gsems[buf]
            ).wait()

        def store_start(buf, p):
            pltpu.make_async_copy(
                rowbufs[buf], out_hbm.at[pl.ds(bNS, NS), p], ssems[buf]
            ).start()

        def store_wait(buf):
            pltpu.make_async_copy(
                rowbufs[buf], out_hbm.at[pl.ds(bNS, NS), 0], ssems[buf]
            ).wait()

        def ball_query(k, buf):
            qbase = zeros + k * 3
            qx = plsc.load_gather(new_v, [qbase])
            qy = plsc.load_gather(new_v, [qbase + 1])
            qz = plsc.load_gather(new_v, [qbase + 2])
            idxbuf[pl.ds(0, L)] = jnp.full((L,), bN, jnp.int32)

            def cond(jc):
                j, cnt = jc
                return jnp.logical_and(j < n_chunks, cnt < NS)

            def one(j, cnt):
                n0 = j * L
                px = xyz_v[0, pl.ds(n0, L)]
                py = xyz_v[1, pl.ds(n0, L)]
                pz = xyz_v[2, pl.ds(n0, L)]
                dx = px - qx
                dy = py - qy
                dz = pz - qz
                d2 = dx * dx + dy * dy + dz * dz
                m = d2 <= r2
                plsc.store_compressed(idxbuf.at[pl.ds(cnt, L)],
                                      lanes + (j * L + bN), mask=m)
                return cnt + plsc.all_reduce_population_count(m)[0]

            def step(jc):
                j, cnt = jc
                cnt = one(j, cnt)
                cnt = one(j + 1, cnt)
                cnt = one(j + 2, cnt)
                cnt = one(j + 3, cnt)
                return j + 4, cnt

            _, total = lax.while_loop(cond, step, (jnp.int32(0), jnp.int32(0)))
            v0 = idxbuf[pl.ds(0, L)]
            v1 = idxbuf[pl.ds(L, L)]
            fvec = jnp.full((L,), v0[0], jnp.int32)
            idxstage[buf, pl.ds(0, L)] = jnp.where(lanes < total, v0, fvec)
            idxstage[buf, pl.ds(L, L)] = jnp.where(lanes + L < total, v1, fvec)

        def per_round(r, carry):
            k0 = r * NB
            for j in range(NB):
                k = k0 + j

                @pl.when(k >= NB)
                def _(j=j):
                    store_wait(j)

                ball_query(k, j)
                gather_start(j)

                pbuf = (j - 1) % NB

                @pl.when(k >= 1)
                def _(pbuf=pbuf, k=k):
                    gather_wait(pbuf)
                    store_start(pbuf, p0 + (k - 1))

            return carry

        lax.fori_loop(0, CPW // NB, per_round, 0)
        lbuf = (CPW - 1) % NB
        gather_wait(lbuf)
        store_start(lbuf, p0 + (CPW - 1))
        for buf in range(NB):
            store_wait(buf)

    fused = pl.kernel(
        body,
        out_type=jax.ShapeDtypeStruct((B * NS, NP, DP), jnp.float32),
        mesh=mesh,
        compiler_params=pltpu.CompilerParams(needs_layout_passes=False),
        scratch_types=[
            pltpu.VMEM((3, N), jnp.float32),
            pltpu.VMEM((CPW * 3,), jnp.float32),
            pltpu.VMEM((8 * L,), jnp.int32),
            pltpu.VMEM((NB, 2 * L), jnp.int32),
            [pltpu.VMEM((NS, DP), jnp.float32) for _ in range(NB)],
            [pltpu.SemaphoreType.DMA for _ in range(NB)],
            [pltpu.SemaphoreType.DMA for _ in range(NB)],
        ],
    )
    return fused(xyz, new_xyz.reshape(B, NP * 3), table)


def _untranspose(gathered, new_xyz_embed, NP, D):
    TOT, DP = gathered.shape
    B, _, EMB = new_xyz_embed.shape
    NS = _NSAMPLE
    PT = 256
    n_blk = NP // PT
    g3 = gathered.reshape(B * NS, NP, DP)

    def body(g_ref, emb_ref, out_ref):
        et = emb_ref[0].T
        for s in range(NS):
            gt = g_ref[s, :, :D].T
            out_ref[0, :EMB, s, :] = gt[:EMB, :] - et
            out_ref[0, EMB:, s, :] = gt[EMB:, :]

    out3 = pl.pallas_call(
        body,
        grid=(B, n_blk),
        in_specs=[
            pl.BlockSpec((NS, PT, DP), lambda b, i: (b, i, 0)),
            pl.BlockSpec((1, PT, EMB), lambda b, i: (b, i, 0)),
        ],
        out_specs=pl.BlockSpec((1, D, NS, PT), lambda b, i: (b, 0, 0, i)),
        out_shape=jax.ShapeDtypeStruct((B, D, NS, NP), jnp.float32),
    )(g3, new_xyz_embed)
    return jnp.swapaxes(out3, 2, 3)


def kernel(xyz, xyz_embed, new_xyz, new_xyz_embed, features):
    NP = new_xyz.shape[1]
    D = xyz_embed.shape[2] + features.shape[1]
    table, planes = _build_table(xyz_embed, features, xyz)
    g3 = _bq_gather_sc(planes, new_xyz, table)
    gathered = g3.reshape(g3.shape[0] * NP, g3.shape[2])
    return _untranspose(gathered, new_xyz_embed, NP, D)


# speedup vs baseline: 21.2654x; 1.0277x over previous
import jax
import jax.numpy as jnp
from jax import lax
from jax.experimental import pallas as pl
from jax.experimental.pallas import tpu as pltpu
from jax.experimental.pallas import tpu_sc as plsc

_RADIUS = 0.2
_NSAMPLE = 32


def _build_table(xyz_embed, features, xyz):
    B, N, EMB = xyz_embed.shape
    C = features.shape[1]
    D = EMB + C
    DP = 384
    TN = 512
    n_blk = N // TN

    def body(emb_ref, feat_ref, xyz_ref, out_ref, pl_ref):
        pad = jnp.zeros((TN, DP - D), jnp.float32)
        out_ref[...] = jnp.concatenate([emb_ref[0], feat_ref[0].T, pad], axis=1)
        pl_ref[0] = xyz_ref[0].T

    return pl.pallas_call(
        body,
        grid=(B, n_blk),
        in_specs=[
            pl.BlockSpec((1, TN, EMB), lambda b, i: (b, i, 0)),
            pl.BlockSpec((1, C, TN), lambda b, i: (b, 0, i)),
            pl.BlockSpec((1, TN, 3), lambda b, i: (b, i, 0)),
        ],
        out_specs=[
            pl.BlockSpec((TN, DP), lambda b, i: (b * n_blk + i, 0)),
            pl.BlockSpec((1, 3, TN), lambda b, i: (b, 0, i)),
        ],
        out_shape=[
            jax.ShapeDtypeStruct((B * N, DP), jnp.float32),
            jax.ShapeDtypeStruct((B, 3, N), jnp.float32),
        ],
    )(xyz_embed, features, xyz)


def _bq_gather_sc(xyz, new_xyz, table):
    B, _, N = xyz.shape
    NP = new_xyz.shape[1]
    NS = _NSAMPLE
    DP = table.shape[1]
    r2 = _RADIUS * _RADIUS

    info = plsc.get_sparse_core_info()
    NC, NSUB, L = info.num_cores, info.num_subcores, info.num_lanes
    NW = NC * NSUB
    CPW = (B * NP) // NW
    n_chunks = N // L
    NB = 4

    mesh = plsc.VectorSubcoreMesh(core_axis_name="c", subcore_axis_name="s")

    def body(xyz_hbm, new_hbm, table_hbm, out_hbm, xyz_v, new_v, idxbuf,
             idxstage, rowbufs, gsems, ssems):
        cid = lax.axis_index("c")
        sid = lax.axis_index("s")
        wid = sid * NC + cid
        g0 = wid * CPW
        b = g0 // NP
        p0 = g0 % NP
        pltpu.sync_copy(xyz_hbm.at[b], xyz_v)
        pltpu.sync_copy(new_hbm.at[b, pl.ds(p0 * 3, CPW * 3)], new_v)
        bN = b * N
        bNS = b * NS
        lanes = lax.broadcasted_iota(jnp.int32, (L,), 0)
        zeros = jnp.zeros((L,), jnp.int32)

        def gather_start(buf):
            pltpu.make_async_copy(
                table_hbm.at[idxstage.at[buf]], rowbufs[buf], gsems[buf]
            ).start()

        def gather_wait(buf):
            pltpu.make_async_copy(
                table_hbm.at[idxstage.at[0]], rowbufs[buf], gsems[buf]
            ).wait()

        def store_start(buf, p):
            pltpu.make_async_copy(
                rowbufs[buf], out_hbm.at[pl.ds(bNS, NS), p], ssems[buf]
            ).start()

        def store_wait(buf):
            pltpu.make_async_copy(
                rowbufs[buf], out_hbm.at[pl.ds(bNS, NS), 0], ssems[buf]
            ).wait()

        def ball_query(k, buf):
            qbase = zeros + k * 3
            qx = plsc.load_gather(new_v, [qbase])
            qy = plsc.load_gather(new_v, [qbase + 1])
            qz = plsc.load_gather(new_v, [qbase + 2])
            idxbuf[pl.ds(0, L)] = jnp.full((L,), bN, jnp.int32)

            def cond(jc):
                j, cnt = jc
                return jnp.logical_and(j < n_chunks, cnt < NS)

            def one(j, cnt):
                n0 = j * L
                px = xyz_v[0, pl.ds(n0, L)]
                py = xyz_v[1, pl.ds(n0, L)]
                pz = xyz_v[2, pl.ds(n0, L)]
                dx = px - qx
                dy = py - qy
                dz = pz - qz
                d2 = dx * dx + dy * dy + dz * dz
                m = d2 <= r2
                plsc.store_compressed(idxbuf.at[pl.ds(cnt, L)],
                                      lanes + (j * L + bN), mask=m)
                return cnt + plsc.all_reduce_population_count(m)[0]

            def step(jc):
                j, cnt = jc
                for u in range(8):
                    cnt = one(j + u, cnt)
                return j + 8, cnt

            _, total = lax.while_loop(cond, step, (jnp.int32(0), jnp.int32(0)))
            v0 = idxbuf[pl.ds(0, L)]
            v1 = idxbuf[pl.ds(L, L)]
            fvec = jnp.full((L,), v0[0], jnp.int32)
            idxstage[buf, pl.ds(0, L)] = jnp.where(lanes < total, v0, fvec)
            idxstage[buf, pl.ds(L, L)] = jnp.where(lanes + L < total, v1, fvec)

        def per_round(r, carry):
            k0 = r * NB
            for j in range(NB):
                k = k0 + j

                @pl.when(k >= NB)
                def _(j=j):
                    store_wait(j)

                ball_query(k, j)
                gather_start(j)

                pbuf = (j - 1) % NB

                @pl.when(k >= 1)
                def _(pbuf=pbuf, k=k):
                    gather_wait(pbuf)
                    store_start(pbuf, p0 + (k - 1))

            return carry

        lax.fori_loop(0, CPW // NB, per_round, 0)
        lbuf = (CPW - 1) % NB
        gather_wait(lbuf)
        store_start(lbuf, p0 + (CPW - 1))
        for buf in range(NB):
            store_wait(buf)

    fused = pl.kernel(
        body,
        out_type=jax.ShapeDtypeStruct((B * NS, NP, DP), jnp.float32),
        mesh=mesh,
        compiler_params=pltpu.CompilerParams(needs_layout_passes=False),
        scratch_types=[
            pltpu.VMEM((3, N), jnp.float32),
            pltpu.VMEM((CPW * 3,), jnp.float32),
            pltpu.VMEM((12 * L,), jnp.int32),
            pltpu.VMEM((NB, 2 * L), jnp.int32),
            [pltpu.VMEM((NS, DP), jnp.float32) for _ in range(NB)],
            [pltpu.SemaphoreType.DMA for _ in range(NB)],
            [pltpu.SemaphoreType.DMA for _ in range(NB)],
        ],
    )
    return fused(xyz, new_xyz.reshape(B, NP * 3), table)


def _untranspose(gathered, new_xyz_embed, NP, D):
    TOT, DP = gathered.shape
    B, _, EMB = new_xyz_embed.shape
    NS = _NSAMPLE
    PT = 256
    n_blk = NP // PT
    g3 = gathered.reshape(B * NS, NP, DP)

    def body(g_ref, emb_ref, out_ref):
        et = emb_ref[0].T
        for s in range(NS):
            gt = g_ref[s, :, :D].T
            out_ref[0, :EMB, s, :] = gt[:EMB, :] - et
            out_ref[0, EMB:, s, :] = gt[EMB:, :]

    out3 = pl.pallas_call(
        body,
        grid=(B, n_blk),
        in_specs=[
            pl.BlockSpec((NS, PT, DP), lambda b, i: (b, i, 0)),
            pl.BlockSpec((1, PT, EMB), lambda b, i: (b, i, 0)),
        ],
        out_specs=pl.BlockSpec((1, D, NS, PT), lambda b, i: (b, 0, 0, i)),
        out_shape=jax.ShapeDtypeStruct((B, D, NS, NP), jnp.float32),
    )(g3, new_xyz_embed)
    return jnp.swapaxes(out3, 2, 3)


def kernel(xyz, xyz_embed, new_xyz, new_xyz_embed, features):
    NP = new_xyz.shape[1]
    D = xyz_embed.shape[2] + features.shape[1]
    table, planes = _build_table(xyz_embed, features, xyz)
    g3 = _bq_gather_sc(planes, new_xyz, table)
    gathered = g3.reshape(g3.shape[0] * NP, g3.shape[2])
    return _untranspose(gathered, new_xyz_embed, NP, D)


# speedup vs baseline: 22.0319x; 1.0360x over previous
import jax
import jax.numpy as jnp
from jax import lax
from jax.experimental import pallas as pl
from jax.experimental.pallas import tpu as pltpu
from jax.experimental.pallas import tpu_sc as plsc

_RADIUS = 0.2
_NSAMPLE = 32


def _build_table(xyz_embed, features, xyz):
    B, N, EMB = xyz_embed.shape
    C = features.shape[1]
    D = EMB + C
    DP = 384
    TN = 2048
    n_blk = N // TN

    def body(emb_ref, feat_ref, xyz_ref, out_ref, pl_ref):
        pad = jnp.zeros((TN, DP - D), jnp.float32)
        out_ref[...] = jnp.concatenate([emb_ref[0], feat_ref[0].T, pad], axis=1)
        pl_ref[0] = xyz_ref[0].T

    return pl.pallas_call(
        body,
        grid=(B, n_blk),
        in_specs=[
            pl.BlockSpec((1, TN, EMB), lambda b, i: (b, i, 0)),
            pl.BlockSpec((1, C, TN), lambda b, i: (b, 0, i)),
            pl.BlockSpec((1, TN, 3), lambda b, i: (b, i, 0)),
        ],
        out_specs=[
            pl.BlockSpec((TN, DP), lambda b, i: (b * n_blk + i, 0)),
            pl.BlockSpec((1, 3, TN), lambda b, i: (b, 0, i)),
        ],
        out_shape=[
            jax.ShapeDtypeStruct((B * N, DP), jnp.float32),
            jax.ShapeDtypeStruct((B, 3, N), jnp.float32),
        ],
    )(xyz_embed, features, xyz)


def _bq_gather_sc(xyz, new_xyz, table):
    B, _, N = xyz.shape
    NP = new_xyz.shape[1]
    NS = _NSAMPLE
    DP = table.shape[1]
    r2 = _RADIUS * _RADIUS

    info = plsc.get_sparse_core_info()
    NC, NSUB, L = info.num_cores, info.num_subcores, info.num_lanes
    NW = NC * NSUB
    CPW = (B * NP) // NW
    n_chunks = N // L
    NB = 4

    mesh = plsc.VectorSubcoreMesh(core_axis_name="c", subcore_axis_name="s")

    def body(xyz_hbm, new_hbm, table_hbm, out_hbm, xyz_v, new_v, idxbuf,
             idxstage, rowbufs, gsems, ssems):
        cid = lax.axis_index("c")
        sid = lax.axis_index("s")
        wid = sid * NC + cid
        g0 = wid * CPW
        b = g0 // NP
        p0 = g0 % NP
        pltpu.sync_copy(xyz_hbm.at[b], xyz_v)
        pltpu.sync_copy(new_hbm.at[b, pl.ds(p0 * 3, CPW * 3)], new_v)
        bN = b * N
        bNS = b * NS
        lanes = lax.broadcasted_iota(jnp.int32, (L,), 0)
        zeros = jnp.zeros((L,), jnp.int32)

        def gather_start(buf):
            pltpu.make_async_copy(
                table_hbm.at[idxstage.at[buf]], rowbufs[buf], gsems[buf]
            ).start()

        def gather_wait(buf):
            pltpu.make_async_copy(
                table_hbm.at[idxstage.at[0]], rowbufs[buf], gsems[buf]
            ).wait()

        def store_start(buf, p):
            pltpu.make_async_copy(
                rowbufs[buf], out_hbm.at[pl.ds(bNS, NS), p], ssems[buf]
            ).start()

        def store_wait(buf):
            pltpu.make_async_copy(
                rowbufs[buf], out_hbm.at[pl.ds(bNS, NS), 0], ssems[buf]
            ).wait()

        def ball_query(k, buf):
            qbase = zeros + k * 3
            qx = plsc.load_gather(new_v, [qbase])
            qy = plsc.load_gather(new_v, [qbase + 1])
            qz = plsc.load_gather(new_v, [qbase + 2])
            idxbuf[pl.ds(0, L)] = jnp.full((L,), bN, jnp.int32)

            def cond(jc):
                j, cnt = jc
                return jnp.logical_and(j < n_chunks, cnt < NS)

            def one(j, cnt):
                n0 = j * L
                px = xyz_v[0, pl.ds(n0, L)]
                py = xyz_v[1, pl.ds(n0, L)]
                pz = xyz_v[2, pl.ds(n0, L)]
                dx = px - qx
                dy = py - qy
                dz = pz - qz
                d2 = dx * dx + dy * dy + dz * dz
                m = d2 <= r2
                plsc.store_compressed(idxbuf.at[pl.ds(cnt, L)],
                                      lanes + (j * L + bN), mask=m)
                return cnt + plsc.all_reduce_population_count(m)[0]

            def step(jc):
                j, cnt = jc
                for u in range(8):
                    cnt = one(j + u, cnt)
                return j + 8, cnt

            _, total = lax.while_loop(cond, step, (jnp.int32(0), jnp.int32(0)))
            v0 = idxbuf[pl.ds(0, L)]
            v1 = idxbuf[pl.ds(L, L)]
            fvec = jnp.full((L,), v0[0], jnp.int32)
            idxstage[buf, pl.ds(0, L)] = jnp.where(lanes < total, v0, fvec)
            idxstage[buf, pl.ds(L, L)] = jnp.where(lanes + L < total, v1, fvec)

        def per_round(r, carry):
            k0 = r * NB
            for j in range(NB):
                k = k0 + j

                @pl.when(k >= NB)
                def _(j=j):
                    store_wait(j)

                ball_query(k, j)
                gather_start(j)

                pbuf = (j - 1) % NB

                @pl.when(k >= 1)
                def _(pbuf=pbuf, k=k):
                    gather_wait(pbuf)
                    store_start(pbuf, p0 + (k - 1))

            return carry

        lax.fori_loop(0, CPW // NB, per_round, 0)
        lbuf = (CPW - 1) % NB
        gather_wait(lbuf)
        store_start(lbuf, p0 + (CPW - 1))
        for buf in range(NB):
            store_wait(buf)

    fused = pl.kernel(
        body,
        out_type=jax.ShapeDtypeStruct((B * NS, NP, DP), jnp.float32),
        mesh=mesh,
        compiler_params=pltpu.CompilerParams(needs_layout_passes=False),
        scratch_types=[
            pltpu.VMEM((3, N), jnp.float32),
            pltpu.VMEM((CPW * 3,), jnp.float32),
            pltpu.VMEM((12 * L,), jnp.int32),
            pltpu.VMEM((NB, 2 * L), jnp.int32),
            [pltpu.VMEM((NS, DP), jnp.float32) for _ in range(NB)],
            [pltpu.SemaphoreType.DMA for _ in range(NB)],
            [pltpu.SemaphoreType.DMA for _ in range(NB)],
        ],
    )
    return fused(xyz, new_xyz.reshape(B, NP * 3), table)


def _untranspose(gathered, new_xyz_embed, NP, D):
    TOT, DP = gathered.shape
    B, _, EMB = new_xyz_embed.shape
    NS = _NSAMPLE
    PT = 256
    n_blk = NP // PT
    g3 = gathered.reshape(B * NS, NP, DP)

    def body(g_ref, emb_ref, out_ref):
        et = emb_ref[0].T
        for s in range(NS):
            gt = g_ref[s, :, :D].T
            out_ref[0, :EMB, s, :] = gt[:EMB, :] - et
            out_ref[0, EMB:, s, :] = gt[EMB:, :]

    out3 = pl.pallas_call(
        body,
        grid=(B, n_blk),
        in_specs=[
            pl.BlockSpec((NS, PT, DP), lambda b, i: (b, i, 0)),
            pl.BlockSpec((1, PT, EMB), lambda b, i: (b, i, 0)),
        ],
        out_specs=pl.BlockSpec((1, D, NS, PT), lambda b, i: (b, 0, 0, i)),
        out_shape=jax.ShapeDtypeStruct((B, D, NS, NP), jnp.float32),
    )(g3, new_xyz_embed)
    return jnp.swapaxes(out3, 2, 3)


def kernel(xyz, xyz_embed, new_xyz, new_xyz_embed, features):
    NP = new_xyz.shape[1]
    D = xyz_embed.shape[2] + features.shape[1]
    table, planes = _build_table(xyz_embed, features, xyz)
    g3 = _bq_gather_sc(planes, new_xyz, table)
    gathered = g3.reshape(g3.shape[0] * NP, g3.shape[2])
    return _untranspose(gathered, new_xyz_embed, NP, D)
